# baseline (speedup 1.0000x reference)
import functools

import jax
import jax.numpy as jnp
from jax import lax
from jax.experimental import pallas as pl
from jax.experimental.pallas import tpu as pltpu

V = 8192
C = 256
M = 8192
BETA = 0.25

BM = 512
BN = 2048
NM = M // BM
NN = V // BN

_I32_MAX = 2**31 - 1


def _bf16_rne(x):
    return x.astype(jnp.bfloat16).astype(jnp.float32)


def _argmin_body(fl2_ref, flsq_ref, cb_ref, tok_ref, dmin_ref, acc_ref):
    n = pl.program_id(1)
    fl2 = fl2_ref[...]
    cb = cb_ref[...]
    cbb = cb.astype(jnp.bfloat16)
    s = lax.dot_general(fl2, cbb, (((1,), (1,)), ((), ())),
                        preferred_element_type=jnp.float32)
    zz = jnp.sum(flsq_ref[...], axis=1, keepdims=True)
    cc = jnp.sum(cb * cb, axis=1)
    d2 = (zz - s) + cc[None, :]
    bmin = jnp.min(d2, axis=1)
    col = lax.broadcasted_iota(jnp.int32, (BM, BN), 1) + n * BN
    bidx = jnp.min(jnp.where(d2 == bmin[:, None], col, _I32_MAX), axis=1)

    @pl.when(n == 0)
    def _init():
        acc_ref[...] = _bf16_rne(bmin)
        dmin_ref[...] = bmin
        tok_ref[...] = bidx

    @pl.when(n > 0)
    def _fold():
        run = acc_ref[...]
        better = bmin < run
        new_acc = jnp.where(better, bmin, run)
        new_acc = jnp.where((n % 2) == 0, _bf16_rne(new_acc), new_acc)
        acc_ref[...] = new_acc
        dmin_ref[...] = jnp.where(better, bmin, dmin_ref[...])
        tok_ref[...] = jnp.where(better, bidx, tok_ref[...])


def _tokens_and_dmin(fl2_bf16, flsq, codebook):
    return pl.pallas_call(
        _argmin_body,
        grid=(NM, NN),
        in_specs=[pl.BlockSpec((BM, C), lambda m, n: (m, 0)),
                  pl.BlockSpec((BM, C), lambda m, n: (m, 0)),
                  pl.BlockSpec((BN, C), lambda m, n: (n, 0))],
        out_specs=[pl.BlockSpec((BM,), lambda m, n: (m,)),
                   pl.BlockSpec((BM,), lambda m, n: (m,)),
                   pl.BlockSpec((BM,), lambda m, n: (m,))],
        out_shape=[jax.ShapeDtypeStruct((M,), jnp.int32),
                   jax.ShapeDtypeStruct((M,), jnp.float32),
                   jax.ShapeDtypeStruct((M,), jnp.float32)],
        compiler_params=pltpu.CompilerParams(
            dimension_semantics=("parallel", "arbitrary")),
    )(fl2_bf16, flsq, codebook)[:2]


def _make_sc_gather():
    from jax.experimental.pallas import tpu_sc as plsc

    info = plsc.get_sparse_core_info()
    NW = info.num_cores * info.num_subcores
    b_per_w = M // NW
    CH = 128
    NCH = b_per_w // CH
    mesh = plsc.VectorSubcoreMesh(core_axis_name="c", subcore_axis_name="s")

    @functools.partial(
        pl.kernel, mesh=mesh,
        out_type=jax.ShapeDtypeStruct((M, C), jnp.float32),
        scratch_types=[
            pltpu.VMEM((NCH, CH), jnp.int32),
            pltpu.VMEM((CH, C), jnp.float32),
            pltpu.SemaphoreType.DMA,
        ],
    )
    def sc_gather(table_hbm, idx_hbm, out_hbm, idx_v, rows_v, sem):
        wid = lax.axis_index("s") * info.num_cores + lax.axis_index("c")
        base = wid * b_per_w
        pltpu.sync_copy(idx_hbm.at[pl.ds(wid * NCH, NCH)], idx_v)
        for k in range(NCH):
            pltpu.async_copy(table_hbm.at[idx_v.at[k]], rows_v, sem).wait()
            pltpu.sync_copy(rows_v, out_hbm.at[pl.ds(base + k * CH, CH)])

    return sc_gather


_sc_gather = None


def _gather_rows(codebook, tokens):
    global _sc_gather
    if _sc_gather is None:
        _sc_gather = _make_sc_gather()
    return _sc_gather(codebook, tokens.reshape(M // 128, 128))


def _scalars_body(tok_ref, dmin_ref, loss_ref, perp_ref):
    toks = tok_ref[...].reshape(8, 1024)
    bins = lax.broadcasted_iota(jnp.int32, (1024, V), 1)
    counts = jnp.zeros((V,), jnp.float32)
    for i in range(8):
        eq = (toks[i, :, None] == bins).astype(jnp.float32)
        counts = counts + jnp.sum(eq, axis=0)
    p = counts / jnp.sum(counts)
    perp = jnp.exp(-jnp.sum(p * jnp.log(p + 1e-10)))
    loss = (1.0 + BETA) * jnp.sum(dmin_ref[...]) / (M * C)
    loss_ref[0] = loss
    perp_ref[0] = perp


def _scalars(tokens, dmin):
    loss, perp = pl.pallas_call(
        _scalars_body,
        out_shape=[jax.ShapeDtypeStruct((1,), jnp.float32),
                   jax.ShapeDtypeStruct((1,), jnp.float32)],
        out_specs=[pl.BlockSpec(memory_space=pltpu.SMEM),
                   pl.BlockSpec(memory_space=pltpu.SMEM)],
    )(tokens, dmin)
    return loss[0], perp[0]


def kernel(z, codebook):
    b, c, h, w = z.shape
    flatten = jnp.transpose(z, (0, 2, 3, 1)).reshape(-1, c)
    fl2 = (2.0 * flatten).astype(jnp.bfloat16)
    flsq = flatten * flatten
    tokens, dmin = _tokens_and_dmin(fl2, flsq, codebook)

    rows = _gather_rows(codebook, tokens)
    quantized = rows.reshape(b, h, w, c).transpose(0, 3, 1, 2)

    loss, perplexity = _scalars(tokens, dmin)
    tokens_out = tokens.reshape(b, h, w)
    quantized_st = z + jax.lax.stop_gradient(quantized - z)
    return (quantized_st, tokens_out, loss, perplexity)


# speedup vs baseline: 1.0802x; 1.0802x over previous
import functools

import jax
import jax.numpy as jnp
from jax import lax
from jax.experimental import pallas as pl
from jax.experimental.pallas import tpu as pltpu

V = 8192
C = 256
M = 8192
BETA = 0.25

BM = 1024
BN = 2048
NM = M // BM
NN = V // BN

_I32_MAX = 2**31 - 1


def _bf16_rne(x):
    return x.astype(jnp.bfloat16).astype(jnp.float32)


def _argmin_body(fl2_ref, flsq_ref, cb_ref, tok_ref, dmin_ref, acc_ref):
    n = pl.program_id(1)
    fl2 = fl2_ref[...]
    cb = cb_ref[...]
    cbb = cb.astype(jnp.bfloat16)
    s = lax.dot_general(fl2, cbb, (((1,), (1,)), ((), ())),
                        preferred_element_type=jnp.float32)
    zz = jnp.sum(flsq_ref[...], axis=1, keepdims=True)
    cc = jnp.sum(cb * cb, axis=1)
    d2 = (zz - s) + cc[None, :]
    bmin = jnp.min(d2, axis=1)
    col = lax.broadcasted_iota(jnp.int32, (BM, BN), 1) + n * BN
    bidx = jnp.min(jnp.where(d2 == bmin[:, None], col, _I32_MAX), axis=1)

    @pl.when(n == 0)
    def _init():
        acc_ref[...] = _bf16_rne(bmin)
        dmin_ref[...] = bmin
        tok_ref[...] = bidx

    @pl.when(n > 0)
    def _fold():
        run = acc_ref[...]
        better = bmin < run
        new_acc = jnp.where(better, bmin, run)
        new_acc = jnp.where((n % 2) == 0, _bf16_rne(new_acc), new_acc)
        acc_ref[...] = new_acc
        dmin_ref[...] = jnp.where(better, bmin, dmin_ref[...])
        tok_ref[...] = jnp.where(better, bidx, tok_ref[...])


def _tokens_and_dmin(fl2_bf16, flsq, codebook):
    return pl.pallas_call(
        _argmin_body,
        grid=(NM, NN),
        in_specs=[pl.BlockSpec((BM, C), lambda m, n: (m, 0)),
                  pl.BlockSpec((BM, C), lambda m, n: (m, 0)),
                  pl.BlockSpec((BN, C), lambda m, n: (n, 0))],
        out_specs=[pl.BlockSpec((BM,), lambda m, n: (m,)),
                   pl.BlockSpec((BM,), lambda m, n: (m,)),
                   pl.BlockSpec((BM,), lambda m, n: (m,))],
        out_shape=[jax.ShapeDtypeStruct((M,), jnp.int32),
                   jax.ShapeDtypeStruct((M,), jnp.float32),
                   jax.ShapeDtypeStruct((M,), jnp.float32)],
        compiler_params=pltpu.CompilerParams(
            dimension_semantics=("parallel", "arbitrary")),
    )(fl2_bf16, flsq, codebook)[:2]


def _make_sc_gather():
    from jax.experimental.pallas import tpu_sc as plsc

    info = plsc.get_sparse_core_info()
    NW = info.num_cores * info.num_subcores
    b_per_w = M // NW
    CH = 128
    NCH = b_per_w // CH
    mesh = plsc.VectorSubcoreMesh(core_axis_name="c", subcore_axis_name="s")

    @functools.partial(
        pl.kernel, mesh=mesh,
        out_type=jax.ShapeDtypeStruct((M, C), jnp.float32),
        scratch_types=[
            pltpu.VMEM((NCH, CH), jnp.int32),
            pltpu.VMEM((CH, C), jnp.float32),
            pltpu.SemaphoreType.DMA,
        ],
    )
    def sc_gather(table_hbm, idx_hbm, out_hbm, idx_v, rows_v, sem):
        wid = lax.axis_index("s") * info.num_cores + lax.axis_index("c")
        base = wid * b_per_w
        pltpu.sync_copy(idx_hbm.at[pl.ds(wid * NCH, NCH)], idx_v)
        for k in range(NCH):
            pltpu.async_copy(table_hbm.at[idx_v.at[k]], rows_v, sem).wait()
            pltpu.sync_copy(rows_v, out_hbm.at[pl.ds(base + k * CH, CH)])

    return sc_gather


_sc_gather = None


def _gather_rows(codebook, tokens):
    global _sc_gather
    if _sc_gather is None:
        _sc_gather = _make_sc_gather()
    return _sc_gather(codebook, tokens.reshape(M // 128, 128))


def _scalars_body(tok_ref, dmin_ref, loss_ref, perp_ref):
    toks = tok_ref[...].reshape(8, 1024)
    bins = lax.broadcasted_iota(jnp.int32, (1024, V), 1)
    counts = jnp.zeros((V,), jnp.float32)
    for i in range(8):
        eq = (toks[i, :, None] == bins).astype(jnp.float32)
        counts = counts + jnp.sum(eq, axis=0)
    p = counts / jnp.sum(counts)
    perp = jnp.exp(-jnp.sum(p * jnp.log(p + 1e-10)))
    loss = (1.0 + BETA) * jnp.sum(dmin_ref[...]) / (M * C)
    loss_ref[0] = loss
    perp_ref[0] = perp


def _scalars(tokens, dmin):
    loss, perp = pl.pallas_call(
        _scalars_body,
        out_shape=[jax.ShapeDtypeStruct((1,), jnp.float32),
                   jax.ShapeDtypeStruct((1,), jnp.float32)],
        out_specs=[pl.BlockSpec(memory_space=pltpu.SMEM),
                   pl.BlockSpec(memory_space=pltpu.SMEM)],
    )(tokens, dmin)
    return loss[0], perp[0]


def kernel(z, codebook):
    b, c, h, w = z.shape
    flatten = jnp.transpose(z, (0, 2, 3, 1)).reshape(-1, c)
    fl2 = (2.0 * flatten).astype(jnp.bfloat16)
    flsq = flatten * flatten
    tokens, dmin = _tokens_and_dmin(fl2, flsq, codebook)

    rows = _gather_rows(codebook, tokens)
    quantized = rows.reshape(b, h, w, c).transpose(0, 3, 1, 2)

    loss, perplexity = _scalars(tokens, dmin)
    tokens_out = tokens.reshape(b, h, w)
    quantized_st = z + jax.lax.stop_gradient(quantized - z)
    return (quantized_st, tokens_out, loss, perplexity)


# speedup vs baseline: 1.1215x; 1.0382x over previous
import functools

import jax
import jax.numpy as jnp
from jax import lax
from jax.experimental import pallas as pl
from jax.experimental.pallas import tpu as pltpu

V = 8192
C = 256
M = 8192
BETA = 0.25

BM = 2048
BN = 2048
NM = M // BM
NN = V // BN

_I32_MAX = 2**31 - 1


def _bf16_rne(x):
    return x.astype(jnp.bfloat16).astype(jnp.float32)


def _argmin_body(fl2_ref, flsq_ref, cb_ref, tok_ref, dmin_ref, acc_ref):
    n = pl.program_id(1)
    fl2 = fl2_ref[...]
    cb = cb_ref[...]
    cbb = cb.astype(jnp.bfloat16)
    s = lax.dot_general(fl2, cbb, (((1,), (1,)), ((), ())),
                        preferred_element_type=jnp.float32)
    zz = jnp.sum(flsq_ref[...], axis=1, keepdims=True)
    cc = jnp.sum(cb * cb, axis=1)
    d2 = (zz - s) + cc[None, :]
    bmin = jnp.min(d2, axis=1)
    col = lax.broadcasted_iota(jnp.int32, (BM, BN), 1) + n * BN
    bidx = jnp.min(jnp.where(d2 == bmin[:, None], col, _I32_MAX), axis=1)

    @pl.when(n == 0)
    def _init():
        acc_ref[...] = _bf16_rne(bmin)
        dmin_ref[...] = bmin
        tok_ref[...] = bidx

    @pl.when(n > 0)
    def _fold():
        run = acc_ref[...]
        better = bmin < run
        new_acc = jnp.where(better, bmin, run)
        new_acc = jnp.where((n % 2) == 0, _bf16_rne(new_acc), new_acc)
        acc_ref[...] = new_acc
        dmin_ref[...] = jnp.where(better, bmin, dmin_ref[...])
        tok_ref[...] = jnp.where(better, bidx, tok_ref[...])


def _tokens_and_dmin(fl2_bf16, flsq, codebook):
    return pl.pallas_call(
        _argmin_body,
        grid=(NM, NN),
        in_specs=[pl.BlockSpec((BM, C), lambda m, n: (m, 0)),
                  pl.BlockSpec((BM, C), lambda m, n: (m, 0)),
                  pl.BlockSpec((BN, C), lambda m, n: (n, 0))],
        out_specs=[pl.BlockSpec((BM,), lambda m, n: (m,)),
                   pl.BlockSpec((BM,), lambda m, n: (m,)),
                   pl.BlockSpec((BM,), lambda m, n: (m,))],
        out_shape=[jax.ShapeDtypeStruct((M,), jnp.int32),
                   jax.ShapeDtypeStruct((M,), jnp.float32),
                   jax.ShapeDtypeStruct((M,), jnp.float32)],
        compiler_params=pltpu.CompilerParams(
            dimension_semantics=("parallel", "arbitrary")),
    )(fl2_bf16, flsq, codebook)[:2]


def _make_sc_gather():
    from jax.experimental.pallas import tpu_sc as plsc

    info = plsc.get_sparse_core_info()
    NW = info.num_cores * info.num_subcores
    b_per_w = M // NW
    CH = 128
    NCH = b_per_w // CH
    mesh = plsc.VectorSubcoreMesh(core_axis_name="c", subcore_axis_name="s")

    @functools.partial(
        pl.kernel, mesh=mesh,
        out_type=jax.ShapeDtypeStruct((M, C), jnp.float32),
        scratch_types=[
            pltpu.VMEM((NCH, CH), jnp.int32),
            pltpu.VMEM((CH, C), jnp.float32),
            pltpu.SemaphoreType.DMA,
        ],
    )
    def sc_gather(table_hbm, idx_hbm, out_hbm, idx_v, rows_v, sem):
        wid = lax.axis_index("s") * info.num_cores + lax.axis_index("c")
        base = wid * b_per_w
        pltpu.sync_copy(idx_hbm.at[pl.ds(wid * NCH, NCH)], idx_v)
        for k in range(NCH):
            pltpu.async_copy(table_hbm.at[idx_v.at[k]], rows_v, sem).wait()
            pltpu.sync_copy(rows_v, out_hbm.at[pl.ds(base + k * CH, CH)])

    return sc_gather


_sc_gather = None


def _gather_rows(codebook, tokens):
    global _sc_gather
    if _sc_gather is None:
        _sc_gather = _make_sc_gather()
    return _sc_gather(codebook, tokens.reshape(M // 128, 128))


def _scalars_body(tok_ref, dmin_ref, loss_ref, perp_ref):
    toks = tok_ref[...].reshape(8, 1024)
    bins = lax.broadcasted_iota(jnp.int32, (1024, V), 1)
    counts = jnp.zeros((V,), jnp.float32)
    for i in range(8):
        eq = (toks[i, :, None] == bins).astype(jnp.float32)
        counts = counts + jnp.sum(eq, axis=0)
    p = counts / jnp.sum(counts)
    perp = jnp.exp(-jnp.sum(p * jnp.log(p + 1e-10)))
    loss = (1.0 + BETA) * jnp.sum(dmin_ref[...]) / (M * C)
    loss_ref[0] = loss
    perp_ref[0] = perp


def _scalars(tokens, dmin):
    loss, perp = pl.pallas_call(
        _scalars_body,
        out_shape=[jax.ShapeDtypeStruct((1,), jnp.float32),
                   jax.ShapeDtypeStruct((1,), jnp.float32)],
        out_specs=[pl.BlockSpec(memory_space=pltpu.SMEM),
                   pl.BlockSpec(memory_space=pltpu.SMEM)],
    )(tokens, dmin)
    return loss[0], perp[0]


def kernel(z, codebook):
    b, c, h, w = z.shape
    flatten = jnp.transpose(z, (0, 2, 3, 1)).reshape(-1, c)
    fl2 = (2.0 * flatten).astype(jnp.bfloat16)
    flsq = flatten * flatten
    tokens, dmin = _tokens_and_dmin(fl2, flsq, codebook)

    rows = _gather_rows(codebook, tokens)
    quantized = rows.reshape(b, h, w, c).transpose(0, 3, 1, 2)

    loss, perplexity = _scalars(tokens, dmin)
    tokens_out = tokens.reshape(b, h, w)
    quantized_st = z + jax.lax.stop_gradient(quantized - z)
    return (quantized_st, tokens_out, loss, perplexity)
